# baseline (speedup 1.0000x reference)
import functools

import jax
import jax.numpy as jnp
from jax import lax
from jax.experimental import pallas as pl
from jax.experimental.pallas import tpu as pltpu
from jax.experimental.pallas import tpu_sc as plsc

_B, _L = 16384, 200
_N = _B * _L
_BC = 4096
_UNROLL = 16


def _score_gather(xf, tw, ww):
    mesh = plsc.VectorSubcoreMesh(core_axis_name="c", subcore_axis_name="s")

    @functools.partial(
        pl.kernel,
        out_type=jax.ShapeDtypeStruct((_N,), jnp.float32),
        mesh=mesh,
        scratch_types=[
            pltpu.VMEM((8, 16), jnp.float32),
            pltpu.VMEM((8, 16), jnp.float32),
            pltpu.SemaphoreType.DMA,
        ],
    )
    def run(x_hbm, t_hbm, w_hbm, o_hbm, t_v, w_v, sem):
        pltpu.async_copy(t_hbm, t_v, sem).wait()
        pltpu.async_copy(w_hbm, w_v, sem).wait()

        scores = t_v[0] * w_v[0]
        for d in range(1, 6):
            scores = scores + t_v[d] * w_v[d]

        dnums = lax.GatherDimensionNumbers(
            offset_dims=(), collapsed_slice_dims=(0,), start_index_map=(0,)
        )

        def body(x_vmem, o_vmem):
            @plsc.parallel_loop(0, _BC, step=16, unroll=_UNROLL)
            def _(j):
                idx = x_vmem[0, pl.ds(j, 16)]
                o_vmem[pl.ds(j, 16)] = lax.gather(
                    scores,
                    idx[:, None],
                    dnums,
                    slice_sizes=(1,),
                    mode=lax.GatherScatterMode.PROMISE_IN_BOUNDS,
                )

        nc = _B // _BC
        pltpu.emit_pipeline(
            body,
            grid=(_L, nc),
            in_specs=[pl.BlockSpec((1, _BC), lambda i, j: (i, j))],
            out_specs=[pl.BlockSpec((_BC,), lambda i, j: (i * nc + j,))],
            core_axis_name=("c", "s"),
            dimension_semantics=(pltpu.PARALLEL, pltpu.PARALLEL),
        )(x_hbm, o_hbm)

    return run(xf, tw, ww)


def kernel(x, table, W, b):
    xf = jnp.transpose(x)
    tw = (
        jnp.zeros((8, 16), jnp.float32)
        .at[:5, :10]
        .set(table.T)
        .at[5, :]
        .set(1.0)
    )
    ww = (
        jnp.zeros((8, 16), jnp.float32)
        .at[:5, :]
        .set(jnp.broadcast_to(W.reshape(5, 1), (5, 16)))
        .at[5, :]
        .set(b[0])
    )
    out = _score_gather(xf, tw, ww)
    return out.reshape(_L, _B, 1).transpose(1, 0, 2)


# speedup vs baseline: 1.1323x; 1.1323x over previous
import functools

import jax
import jax.numpy as jnp
from jax import lax
from jax.experimental import pallas as pl
from jax.experimental.pallas import tpu as pltpu
from jax.experimental.pallas import tpu_sc as plsc

_B, _L = 16384, 200
_N = _B * _L
_BC = 8192
_UNROLL = 16


def _score_gather(xf, tw, ww):
    mesh = plsc.VectorSubcoreMesh(core_axis_name="c", subcore_axis_name="s")

    @functools.partial(
        pl.kernel,
        out_type=jax.ShapeDtypeStruct((_N,), jnp.float32),
        mesh=mesh,
        scratch_types=[
            pltpu.VMEM((8, 16), jnp.float32),
            pltpu.VMEM((8, 16), jnp.float32),
            pltpu.SemaphoreType.DMA,
        ],
    )
    def run(x_hbm, t_hbm, w_hbm, o_hbm, t_v, w_v, sem):
        pltpu.async_copy(t_hbm, t_v, sem).wait()
        pltpu.async_copy(w_hbm, w_v, sem).wait()

        scores = t_v[0] * w_v[0]
        for d in range(1, 6):
            scores = scores + t_v[d] * w_v[d]

        dnums = lax.GatherDimensionNumbers(
            offset_dims=(), collapsed_slice_dims=(0,), start_index_map=(0,)
        )

        def body(x_vmem, o_vmem):
            @plsc.parallel_loop(0, _BC, step=16, unroll=_UNROLL)
            def _(j):
                idx = x_vmem[0, pl.ds(j, 16)]
                o_vmem[pl.ds(j, 16)] = lax.gather(
                    scores,
                    idx[:, None],
                    dnums,
                    slice_sizes=(1,),
                    mode=lax.GatherScatterMode.PROMISE_IN_BOUNDS,
                )

        nc = _B // _BC
        pltpu.emit_pipeline(
            body,
            grid=(_L, nc),
            in_specs=[pl.BlockSpec((1, _BC), lambda i, j: (i, j))],
            out_specs=[pl.BlockSpec((_BC,), lambda i, j: (i * nc + j,))],
            core_axis_name=("c", "s"),
            dimension_semantics=(pltpu.PARALLEL, pltpu.PARALLEL),
        )(x_hbm, o_hbm)

    return run(xf, tw, ww)


def kernel(x, table, W, b):
    xf = jnp.transpose(x)
    tw = (
        jnp.zeros((8, 16), jnp.float32)
        .at[:5, :10]
        .set(table.T)
        .at[5, :]
        .set(1.0)
    )
    ww = (
        jnp.zeros((8, 16), jnp.float32)
        .at[:5, :]
        .set(jnp.broadcast_to(W.reshape(5, 1), (5, 16)))
        .at[5, :]
        .set(b[0])
    )
    out = _score_gather(xf, tw, ww)
    return out.reshape(_L, _B, 1).transpose(1, 0, 2)


# speedup vs baseline: 1.1793x; 1.0416x over previous
import functools

import jax
import jax.numpy as jnp
from jax import lax
from jax.experimental import pallas as pl
from jax.experimental.pallas import tpu as pltpu
from jax.experimental.pallas import tpu_sc as plsc

_B, _L = 16384, 200
_N = _B * _L
_BC = 16384
_UNROLL = 16


def _score_gather(xf, tw, ww):
    mesh = plsc.VectorSubcoreMesh(core_axis_name="c", subcore_axis_name="s")

    @functools.partial(
        pl.kernel,
        out_type=jax.ShapeDtypeStruct((_N,), jnp.float32),
        mesh=mesh,
        scratch_types=[
            pltpu.VMEM((8, 16), jnp.float32),
            pltpu.VMEM((8, 16), jnp.float32),
            pltpu.SemaphoreType.DMA,
        ],
    )
    def run(x_hbm, t_hbm, w_hbm, o_hbm, t_v, w_v, sem):
        pltpu.async_copy(t_hbm, t_v, sem).wait()
        pltpu.async_copy(w_hbm, w_v, sem).wait()

        scores = t_v[0] * w_v[0]
        for d in range(1, 6):
            scores = scores + t_v[d] * w_v[d]

        dnums = lax.GatherDimensionNumbers(
            offset_dims=(), collapsed_slice_dims=(0,), start_index_map=(0,)
        )

        def body(x_vmem, o_vmem):
            @plsc.parallel_loop(0, _BC, step=16, unroll=_UNROLL)
            def _(j):
                idx = x_vmem[0, pl.ds(j, 16)]
                o_vmem[pl.ds(j, 16)] = lax.gather(
                    scores,
                    idx[:, None],
                    dnums,
                    slice_sizes=(1,),
                    mode=lax.GatherScatterMode.PROMISE_IN_BOUNDS,
                )

        nc = _B // _BC
        pltpu.emit_pipeline(
            body,
            grid=(_L, nc),
            in_specs=[pl.BlockSpec((1, _BC), lambda i, j: (i, j))],
            out_specs=[pl.BlockSpec((_BC,), lambda i, j: (i * nc + j,))],
            core_axis_name=("c", "s"),
            dimension_semantics=(pltpu.PARALLEL, pltpu.PARALLEL),
        )(x_hbm, o_hbm)

    return run(xf, tw, ww)


def kernel(x, table, W, b):
    xf = jnp.transpose(x)
    tw = (
        jnp.zeros((8, 16), jnp.float32)
        .at[:5, :10]
        .set(table.T)
        .at[5, :]
        .set(1.0)
    )
    ww = (
        jnp.zeros((8, 16), jnp.float32)
        .at[:5, :]
        .set(jnp.broadcast_to(W.reshape(5, 1), (5, 16)))
        .at[5, :]
        .set(b[0])
    )
    out = _score_gather(xf, tw, ww)
    return out.reshape(_L, _B, 1).transpose(1, 0, 2)
